# baseline (speedup 1.0000x reference)
import jax
import jax.numpy as jnp
from jax import lax
from jax.experimental import pallas as pl
from jax.experimental.pallas import tpu as pltpu
from jax.experimental.pallas import tpu_sc as plsc

B = 4096
ND = 13
NS = 26
D = 32
NW = 32
BPW = B // NW
CPW = NS


def _mm_body(d_ref, w_ref, b_ref, o_ref):
    o_ref[...] = (
        jnp.dot(d_ref[...], w_ref[...], preferred_element_type=jnp.float32)
        + b_ref[...]
    )


def _dense_emb(dense, W, b):
    return pl.pallas_call(
        _mm_body,
        out_shape=jax.ShapeDtypeStruct((B, D), jnp.float32),
    )(dense, W, b.reshape(1, D))


CPAD = 32
SPW = BPW * NS


def _sc_body(idx_hbm, dst_hbm, ddst_hbm, demb_hbm, table_hbm, out_hbm,
             idx_v, dst_v, gbuf, ddst_v, dbuf, sem, sem2):
    wid = lax.axis_index("s") * 2 + lax.axis_index("c")
    pltpu.sync_copy(idx_hbm.at[pl.ds(wid * SPW, SPW)], idx_v)
    pltpu.sync_copy(dst_hbm.at[pl.ds(wid * CPAD, CPAD)], dst_v)
    pltpu.sync_copy(ddst_hbm.at[pl.ds(wid * 8, 8)], ddst_v)
    pltpu.sync_copy(demb_hbm.at[pl.ds(wid * BPW, BPW)], dbuf)
    gathers = []
    for j in range(CPW):
        gathers.append(
            pltpu.async_copy(
                table_hbm.at[idx_v.at[pl.ds(j * 128, 128)]], gbuf.at[j], sem
            )
        )
    for g in gathers:
        g.wait()
    scatters = [pltpu.async_copy(dbuf, out_hbm.at[ddst_v.at[0]], sem2)]
    for j in range(CPW):
        scatters.append(
            pltpu.async_copy(gbuf.at[j], out_hbm.at[dst_v.at[j]], sem2)
        )
    for s in scatters:
        s.wait()


_sc_call = pl.kernel(
    _sc_body,
    out_type=jax.ShapeDtypeStruct((B * 27, D), jnp.float32),
    mesh=plsc.VectorSubcoreMesh(core_axis_name="c", subcore_axis_name="s"),
    compiler_params=pltpu.CompilerParams(use_tc_tiling_on_sc=False),
    scratch_types=[
        pltpu.VMEM((SPW,), jnp.int32),
        pltpu.VMEM((CPAD, 128), jnp.int32),
        pltpu.VMEM((CPW, 128, D), jnp.float32),
        pltpu.VMEM((8, 128), jnp.int32),
        pltpu.VMEM((BPW, D), jnp.float32),
        pltpu.SemaphoreType.DMA,
        pltpu.SemaphoreType.DMA,
    ],
)


def kernel(dense, sparse_idx, table, W, b):
    demb = _dense_emb(dense, W, b)
    idx_flat = sparse_idx.reshape(B * NS)
    p = jnp.arange(B * NS, dtype=jnp.int32)
    dst = (p + p // NS + 1).reshape(NW, CPW, 128)
    dst_pad = jnp.pad(dst, ((0, 0), (0, CPAD - CPW), (0, 0))).reshape(
        NW * CPAD, 128
    )
    ddst = (jnp.arange(B, dtype=jnp.int32) * 27).reshape(NW, 1, 128)
    ddst_pad = jnp.pad(ddst, ((0, 0), (0, 7), (0, 0))).reshape(NW * 8, 128)
    out = _sc_call(idx_flat, dst_pad, ddst_pad, demb, table)
    return out.reshape(B, 27, D)


# speedup vs baseline: 1.6446x; 1.6446x over previous
import jax
import jax.numpy as jnp
from jax import lax
from jax.experimental import pallas as pl
from jax.experimental.pallas import tpu as pltpu
from jax.experimental.pallas import tpu_sc as plsc

B = 4096
VOCAB = 1000000
ND = 13
NS = 26
D = 32
NW = 32
BPW = B // NW
CPW = NS


def _mm_body(d_ref, w_ref, b_ref, o_ref):
    o_ref[...] = (
        jnp.dot(d_ref[...], w_ref[...], preferred_element_type=jnp.float32)
        + b_ref[...]
    )


def _dense_emb(dense, W, b):
    return pl.pallas_call(
        _mm_body,
        out_shape=jax.ShapeDtypeStruct((B, D), jnp.float32),
    )(dense, W, b.reshape(1, D))


RW = 2048
NQ = (VOCAB + 4 * RW - 1) // (4 * RW)
QPAD = NQ * RW


def _repack_body(t_ref, o_ref):
    t = t_ref[...]
    o_ref[...] = jnp.concatenate(
        [t[:, c * RW:(c + 1) * RW].T for c in range(4)], axis=1
    )


def _repack_table(tableT):
    return pl.pallas_call(
        _repack_body,
        grid=(NQ,),
        in_specs=[pl.BlockSpec((D, 4 * RW), lambda j: (0, j))],
        out_specs=pl.BlockSpec((RW, 128), lambda j: (j, 0)),
        out_shape=jax.ShapeDtypeStruct((QPAD, 128), jnp.float32),
    )(tableT)


def _permute_idx(v):
    return 4 * (RW * (v // (4 * RW)) + v % RW) + (v // RW) % 4


CPAD = 32
SPW = BPW * NS


def _sc_body(idx_hbm, dst_hbm, ddst_hbm, demb_hbm, table_hbm, out_hbm,
             idx_v, dst_v, gbuf, ddst_v, dbuf, sem, sem2):
    wid = lax.axis_index("s") * 2 + lax.axis_index("c")
    pltpu.sync_copy(idx_hbm.at[pl.ds(wid * SPW, SPW)], idx_v)
    pltpu.sync_copy(dst_hbm.at[pl.ds(wid * CPAD, CPAD)], dst_v)
    pltpu.sync_copy(ddst_hbm.at[pl.ds(wid * 8, 8)], ddst_v)
    pltpu.sync_copy(demb_hbm.at[pl.ds(wid * BPW, BPW)], dbuf)
    gathers = []
    for j in range(CPW):
        gathers.append(
            pltpu.async_copy(
                table_hbm.at[idx_v.at[pl.ds(j * 128, 128)]], gbuf.at[j], sem
            )
        )
    for g in gathers:
        g.wait()
    scatters = [pltpu.async_copy(dbuf, out_hbm.at[ddst_v.at[0]], sem2)]
    for j in range(CPW):
        scatters.append(
            pltpu.async_copy(gbuf.at[j], out_hbm.at[dst_v.at[j]], sem2)
        )
    for s in scatters:
        s.wait()


_sc_call = pl.kernel(
    _sc_body,
    out_type=jax.ShapeDtypeStruct((B * 27, D), jnp.float32),
    mesh=plsc.VectorSubcoreMesh(core_axis_name="c", subcore_axis_name="s"),
    compiler_params=pltpu.CompilerParams(use_tc_tiling_on_sc=False),
    scratch_types=[
        pltpu.VMEM((SPW,), jnp.int32),
        pltpu.VMEM((CPAD, 128), jnp.int32),
        pltpu.VMEM((CPW, 128, D), jnp.float32),
        pltpu.VMEM((8, 128), jnp.int32),
        pltpu.VMEM((BPW, D), jnp.float32),
        pltpu.SemaphoreType.DMA,
        pltpu.SemaphoreType.DMA,
    ],
)


def kernel(dense, sparse_idx, table, W, b):
    demb = _dense_emb(dense, W, b)
    table_lin = _repack_table(table.T).reshape(4 * QPAD, D)
    idx_flat = _permute_idx(sparse_idx).reshape(B * NS)
    p = jnp.arange(B * NS, dtype=jnp.int32)
    dst = (p + p // NS + 1).reshape(NW, CPW, 128)
    dst_pad = jnp.pad(dst, ((0, 0), (0, CPAD - CPW), (0, 0))).reshape(
        NW * CPAD, 128
    )
    ddst = (jnp.arange(B, dtype=jnp.int32) * 27).reshape(NW, 1, 128)
    ddst_pad = jnp.pad(ddst, ((0, 0), (0, 7), (0, 0))).reshape(NW * 8, 128)
    out = _sc_call(idx_flat, dst_pad, ddst_pad, demb, table_lin)
    return out.reshape(B, 27, D)
